# Initial kernel scaffold; baseline (speedup 1.0000x reference)
import jax
import jax.numpy as jnp
from jax.experimental import pallas as pl


def kernel(embedding, lstm0_wihT, lstm0_whhT, lstm0_b, lstm1_wihT, lstm1_whhT, lstm1_b, waT, ba, wc1T, wc2T, bc, wlT, bl, whpT, bhp, wopT, bop, wa1aT, wa1bT, ba1, wa2T, ba2, tokens):
    raise NotImplementedError("write your pallas kernel here")


# speedup vs baseline: 1.0917x; 1.0917x over previous
import jax
import jax.numpy as jnp
from jax import lax
from jax.experimental import pallas as pl
from jax.experimental.pallas import tpu as pltpu

HID = 512
OUT = 64

_VMEM_LIMIT = 34 * 1024 * 1024
_ANYVMEM = pl.BlockSpec(memory_space=pltpu.MemorySpace.VMEM)


def _lstm_layer_kernel(x_ref, wihT_ref, whhT_ref, b_ref,
                       hseq_ref, hlast_ref, clast_ref, xp_ref):
    nT, nB, in_dim = x_ref.shape
    H = hlast_ref.shape[-1]
    is_bwd = pl.program_id(0) == 1

    x2d = x_ref[...].reshape(nT * nB, in_dim).astype(jnp.bfloat16)
    xp_ref[...] = (jnp.dot(x2d, wihT_ref[...],
                           preferred_element_type=jnp.float32)
                   + b_ref[...]).reshape(nT, nB, 4 * H)

    whh = whhT_ref[...]

    idx0 = jnp.where(is_bwd, nT - 1, 0)
    g0 = xp_ref[idx0]
    c = jax.nn.sigmoid(g0[:, 0 * H:1 * H]) * jnp.tanh(g0[:, 2 * H:3 * H])
    h = jax.nn.sigmoid(g0[:, 3 * H:4 * H]) * jnp.tanh(c)
    hseq_ref[idx0] = h

    for t in range(1, nT):
        idx = jnp.where(is_bwd, nT - 1 - t, t)
        gates = xp_ref[idx] + jnp.dot(h.astype(jnp.bfloat16), whh,
                                      preferred_element_type=jnp.float32)
        i_g = jax.nn.sigmoid(gates[:, 0 * H:1 * H])
        f_g = jax.nn.sigmoid(gates[:, 1 * H:2 * H])
        g_g = jnp.tanh(gates[:, 2 * H:3 * H])
        o_g = jax.nn.sigmoid(gates[:, 3 * H:4 * H])
        c = f_g * c + i_g * g_g
        h = o_g * jnp.tanh(c)
        hseq_ref[idx] = h

    hlast_ref[...] = h
    clast_ref[...] = c


def _lstm_bidir_layer(x_tbf, wihT, whhT, b):
    nT, nB, in_dim = x_tbf.shape
    return pl.pallas_call(
        _lstm_layer_kernel,
        out_shape=(jax.ShapeDtypeStruct((nT, nB, 2 * HID), jnp.float32),
                   jax.ShapeDtypeStruct((2, nB, HID), jnp.float32),
                   jax.ShapeDtypeStruct((2, nB, HID), jnp.float32)),
        grid=(2,),
        in_specs=[
            pl.BlockSpec((nT, nB, in_dim), lambda d: (0, 0, 0)),
            pl.BlockSpec((None, in_dim, 4 * HID), lambda d: (d, 0, 0)),
            pl.BlockSpec((None, HID, 4 * HID), lambda d: (d, 0, 0)),
            pl.BlockSpec((None, 1, 4 * HID), lambda d: (d, 0, 0)),
        ],
        out_specs=(
            pl.BlockSpec((nT, nB, HID), lambda d: (0, 0, d)),
            pl.BlockSpec((None, nB, HID), lambda d: (d, 0, 0)),
            pl.BlockSpec((None, nB, HID), lambda d: (d, 0, 0)),
        ),
        scratch_shapes=[pltpu.VMEM((nT, nB, 4 * HID), jnp.float32)],
        compiler_params=pltpu.CompilerParams(
            dimension_semantics=("parallel",),
            vmem_limit_bytes=_VMEM_LIMIT),
    )(x_tbf, wihT, whhT, b)


def _head_kernel(out_ref, hlast_ref,
                 waT_ref, ba_ref, wc1T_ref, wc2T_ref, bc_ref, wlT_ref, bl_ref,
                 whpT_ref, bhp_ref, wopT_ref, bop_ref,
                 wa1aT_ref, wa1bT_ref, ba1_ref, wa2T_ref, ba2_ref,
                 logits_ref, score_ref, ent_ref, sim_ref, disp_ref):
    eps = 1e-9
    nT, nB, D2 = out_ref.shape
    nO = logits_ref.shape[-1]
    out = out_ref[...]
    out2d_bf = out.reshape(nT * nB, D2).astype(jnp.bfloat16)

    aw = (jnp.dot(out2d_bf, waT_ref[...],
                  preferred_element_type=jnp.float32) + ba_ref[...])
    aw = jnp.exp(aw - jnp.max(aw, axis=-1, keepdims=True))
    aw = aw / jnp.sum(aw, axis=-1, keepdims=True)
    aw3 = aw.reshape(nT, nB, D2)

    ctx = jnp.sum(aw3 * out, axis=0)
    ctxp = jnp.dot(ctx.astype(jnp.bfloat16), wc1T_ref[...],
                   preferred_element_type=jnp.float32)
    comb = (jnp.dot(out2d_bf, wc2T_ref[...],
                    preferred_element_type=jnp.float32).reshape(nT, nB, D2)
            + ctxp[None] + bc_ref[...][None])
    logits2d = (jnp.dot(comb.reshape(nT * nB, D2).astype(jnp.bfloat16),
                        wlT_ref[...], preferred_element_type=jnp.float32)
                + bl_ref[...])
    logits_ref[...] = logits2d.reshape(nT, nB, nO)

    am = jnp.mean(aw3, axis=-1)
    tids = lax.broadcasted_iota(jnp.int32, am.shape, 0)
    rel = jnp.zeros_like(am)
    cur = am
    for _ in range(min(3, nT)):
        mx = jnp.max(cur, axis=0, keepdims=True)
        cand = jnp.where(cur >= mx, tids, nT)
        first = jnp.min(cand, axis=0, keepdims=True)
        pick = (tids == first).astype(jnp.float32)
        rel = rel + pick
        cur = jnp.where(pick > 0, jnp.float32(-1e30), cur)

    h_last = hlast_ref[...]
    out_last = logits2d[(nT - 1) * nB:]
    hp = (jnp.dot(h_last.astype(jnp.bfloat16), whpT_ref[...],
                  preferred_element_type=jnp.float32) + bhp_ref[...])
    op = (jnp.dot(out_last.astype(jnp.bfloat16), wopT_ref[...],
                  preferred_element_type=jnp.float32) + bop_ref[...])

    probs = jnp.exp(op - jnp.max(op, axis=-1, keepdims=True))
    probs = probs / jnp.sum(probs, axis=-1, keepdims=True)
    ent = -jnp.sum(probs * jnp.log(probs + eps), axis=1, keepdims=True)

    dotp = jnp.sum(op * hp, axis=1, keepdims=True)
    n1 = jnp.sqrt(jnp.sum(op * op, axis=1, keepdims=True))
    n2 = jnp.sqrt(jnp.sum(hp * hp, axis=1, keepdims=True))
    sim = dotp / (jnp.maximum(n1, 1e-8) * jnp.maximum(n2, 1e-8))

    ap = jnp.exp(logits2d - jnp.max(logits2d, axis=-1, keepdims=True))
    ap = ap / jnp.sum(ap, axis=-1, keepdims=True)
    t_ent = -jnp.sum(ap * jnp.log(ap + eps), axis=-1,
                     keepdims=True).reshape(nT, nB)
    avg = jnp.mean(ap, axis=-1, keepdims=True).reshape(nT, nB)
    rd_terms = rel * avg * jnp.log(avg + eps)

    ones_t = jnp.ones((nT, 1), jnp.float32)
    dn_t = (((0,), (0,)), ((), ()))
    disp_t = lax.dot_general(t_ent, ones_t, dn_t,
                             precision=lax.Precision.HIGHEST) / nT
    rel_disp = -lax.dot_general(rd_terms, ones_t, dn_t,
                                precision=lax.Precision.HIGHEST)
    disp = (disp_t + rel_disp) * 0.5

    a1 = (jnp.dot(op.astype(jnp.bfloat16), wa1aT_ref[...],
                  preferred_element_type=jnp.float32)
          + jnp.dot(hp.astype(jnp.bfloat16), wa1bT_ref[...],
                    preferred_element_type=jnp.float32)
          + ba1_ref[...])
    a1 = jnp.maximum(a1, 0.0)
    score_pre = (jnp.dot(a1.astype(jnp.bfloat16), wa2T_ref[...],
                         preferred_element_type=jnp.float32) + ba2_ref[...])

    ent_ref[...] = ent
    sim_ref[...] = sim
    disp_ref[...] = disp

    combo = -0.05 * ent + 0.1 * sim + 0.1 * disp
    ones_col = jnp.ones_like(score_pre)
    dn = (((1,), (1,)), ((), ()))
    s = (lax.dot_general(score_pre, ones_col, dn,
                         preferred_element_type=jnp.float32)
         + lax.dot_general(ones_col, combo, dn,
                           preferred_element_type=jnp.float32))
    s = jnp.log(jnp.abs(s) + 1e-9) * jnp.sign(s)
    score_ref[...] = 2.0 * jax.nn.sigmoid(s) - 1.0


def kernel(embedding, lstm0_wihT, lstm0_whhT, lstm0_b,
           lstm1_wihT, lstm1_whhT, lstm1_b,
           waT, ba, wc1T, wc2T, bc, wlT, bl,
           whpT, bhp, wopT, bop,
           wa1aT, wa1bT, ba1, wa2T, ba2, tokens):
    emb_t = jnp.take(embedding, tokens.T, axis=0)

    out0, h0, c0 = _lstm_bidir_layer(emb_t, lstm0_wihT, lstm0_whhT, lstm0_b)
    out1, h1, c1 = _lstm_bidir_layer(out0, lstm1_wihT, lstm1_whhT, lstm1_b)

    h_n = jnp.concatenate([h0, h1], axis=0)
    c_n = jnp.concatenate([c0, c1], axis=0)

    nT, nB, _ = out1.shape
    logits_t, score, ent, sim, disp = pl.pallas_call(
        _head_kernel,
        out_shape=(jax.ShapeDtypeStruct((nT, nB, OUT), jnp.float32),
                   jax.ShapeDtypeStruct((nB, nB), jnp.float32),
                   jax.ShapeDtypeStruct((nB, 1), jnp.float32),
                   jax.ShapeDtypeStruct((nB, 1), jnp.float32),
                   jax.ShapeDtypeStruct((nB, 1), jnp.float32)),
        in_specs=[_ANYVMEM] * 18,
        out_specs=(_ANYVMEM,) * 5,
        compiler_params=pltpu.CompilerParams(vmem_limit_bytes=_VMEM_LIMIT),
    )(out1, h1[1],
      waT, ba, wc1T, wc2T, bc, wlT, bl,
      whpT, bhp, wopT, bop, wa1aT, wa1bT, ba1, wa2T, ba2)

    logits = jnp.transpose(logits_t, (1, 0, 2))
    return (logits, (h_n, c_n), score,
            ent[:, 0], sim[:, 0], disp[:, 0])
